# baseline (speedup 1.0000x reference)
import functools

import jax
import jax.numpy as jnp
from jax import lax
from jax.experimental import pallas as pl
from jax.experimental.pallas import tpu as pltpu
from jax.experimental.pallas import tpu_sc as plsc

_HIDDEN = 4096
_EXPERTS = 64
_TOKENS = 8192
_TM = 512
_NT = _TOKENS // _TM

_NC = 2
_NS = 16
_L = 16
_NW = _NC * _NS
_RPW = _TOKENS // _NW
_NG = _RPW // _L


def _router_tc_kernel(h_ref, w_ref, logits_ref, stat_ref, load_acc, ent_acc):
    i = pl.program_id(0)

    @pl.when(i == 0)
    def _init():
        load_acc[...] = jnp.zeros_like(load_acc)
        ent_acc[0] = 0.0

    h = h_ref[...]
    w = w_ref[...]
    logits = jax.lax.dot_general(
        h, w, (((1,), (1,)), ((), ())),
        preferred_element_type=jnp.float32)
    logits_ref[...] = logits

    m1 = jnp.max(logits, axis=-1, keepdims=True)
    p = jnp.exp(logits - m1)
    denom = jnp.sum(p, axis=-1, keepdims=True)
    probs = p / denom
    load_acc[...] += jnp.sum(probs, axis=0, keepdims=True)
    ent_acc[0] += -jnp.sum(probs * jnp.log(probs + 1e-8))

    @pl.when(i == _NT - 1)
    def _finish():
        load = load_acc[...] / _TOKENS
        mean = jnp.mean(load)
        var = jnp.sum((load - mean) ** 2) / (_EXPERTS - 1)
        stat_ref[0] = var
        stat_ref[1] = ent_acc[0] / _TOKENS


_topk_mesh = plsc.VectorSubcoreMesh(core_axis_name="c", subcore_axis_name="s")


@functools.partial(
    pl.kernel,
    mesh=_topk_mesh,
    out_type=[
        jax.ShapeDtypeStruct((_TOKENS, 2), jnp.int32),
        jax.ShapeDtypeStruct((_TOKENS, 2), jnp.float32),
    ],
    scratch_types=[
        pltpu.VMEM((_RPW, _EXPERTS), jnp.float32),
        pltpu.VMEM((_RPW, 2), jnp.int32),
        pltpu.VMEM((_RPW, 2), jnp.float32),
    ],
    compiler_params=pltpu.CompilerParams(needs_layout_passes=False),
)
def _topk_sc_kernel(logits_hbm, idx_hbm, ew_hbm, lg_v, idx_v, ew_v):
    wid = lax.axis_index("s") * _NC + lax.axis_index("c")
    base = wid * _RPW
    pltpu.sync_copy(logits_hbm.at[pl.ds(base, _RPW)], lg_v)

    zero_i = jnp.zeros((_L,), jnp.int32)
    one_i = jnp.ones((_L,), jnp.int32)
    lane = lax.broadcasted_iota(jnp.int32, (_L,), 0)

    def per_group(g, carry_none):
        rows = g * _L + lane

        def per_col(j, carry):
            m1, i1, m2, i2 = carry
            jv = jnp.full((_L,), j, jnp.int32)
            v = plsc.load_gather(lg_v, [rows, jv])
            gt1 = v > m1
            gt2 = v > m2
            m2n = jnp.where(gt1, m1, jnp.where(gt2, v, m2))
            i2n = jnp.where(gt1, i1, jnp.where(gt2, jv, i2))
            m1n = jnp.where(gt1, v, m1)
            i1n = jnp.where(gt1, jv, i1)
            return (m1n, i1n, m2n, i2n)

        ninf = jnp.full((_L,), -jnp.inf, jnp.float32)
        m1, i1, m2, i2 = lax.fori_loop(
            0, _EXPERTS, per_col, (ninf, zero_i, ninf, zero_i), unroll=8)

        e2 = jnp.exp(m2 - m1)
        d = 1.0 + e2
        w1 = 1.0 / d
        w2 = e2 / d
        plsc.store_scatter(idx_v, [rows, zero_i], i1)
        plsc.store_scatter(idx_v, [rows, one_i], i2)
        plsc.store_scatter(ew_v, [rows, zero_i], w1)
        plsc.store_scatter(ew_v, [rows, one_i], w2)
        return carry_none

    lax.fori_loop(0, _NG, per_group, 0)
    pltpu.sync_copy(idx_v, idx_hbm.at[pl.ds(base, _RPW)])
    pltpu.sync_copy(ew_v, ew_hbm.at[pl.ds(base, _RPW)])


def kernel(hidden_states, router_weight):
    logits, stats = pl.pallas_call(
        _router_tc_kernel,
        grid=(_NT,),
        in_specs=[
            pl.BlockSpec((_TM, _HIDDEN), lambda i: (i, 0)),
            pl.BlockSpec((_EXPERTS, _HIDDEN), lambda i: (0, 0)),
        ],
        out_specs=[
            pl.BlockSpec((_TM, _EXPERTS), lambda i: (i, 0)),
            pl.BlockSpec(memory_space=pltpu.SMEM),
        ],
        out_shape=[
            jax.ShapeDtypeStruct((_TOKENS, _EXPERTS), jnp.float32),
            jax.ShapeDtypeStruct((2,), jnp.float32),
        ],
        scratch_shapes=[
            pltpu.VMEM((1, _EXPERTS), jnp.float32),
            pltpu.SMEM((1,), jnp.float32),
        ],
    )(hidden_states, router_weight)
    idx, ew = _topk_sc_kernel(logits)
    return (logits, idx, ew, stats[0], stats[1])


# speedup vs baseline: 1.3289x; 1.3289x over previous
import jax
import jax.numpy as jnp
from jax.experimental import pallas as pl
from jax.experimental.pallas import tpu as pltpu

_HIDDEN = 4096
_EXPERTS = 64
_TOKENS = 8192
_TM = 512
_NT = _TOKENS // _TM


def _router_kernel(h_ref, w_ref, logits_ref, idx_ref, ew_ref, stat_ref,
                   load_acc, ent_acc):
    i = pl.program_id(0)

    @pl.when(i == 0)
    def _init():
        load_acc[...] = jnp.zeros_like(load_acc)
        ent_acc[0] = 0.0

    h = h_ref[...]
    w = w_ref[...]
    logits = jax.lax.dot_general(
        h, w, (((1,), (1,)), ((), ())),
        preferred_element_type=jnp.float32)
    logits_ref[...] = logits

    lane = jax.lax.broadcasted_iota(jnp.int32, (_TM, _EXPERTS), 1)
    big = jnp.int32(_EXPERTS)
    m1 = jnp.max(logits, axis=-1, keepdims=True)
    i1 = jnp.min(jnp.where(logits == m1, lane, big), axis=-1, keepdims=True)
    masked = jnp.where(lane == i1, -jnp.inf, logits)
    m2 = jnp.max(masked, axis=-1, keepdims=True)
    i2 = jnp.min(jnp.where(masked == m2, lane, big), axis=-1, keepdims=True)
    idx_ref[...] = jnp.concatenate([i1, i2], axis=-1)

    e2 = jnp.exp(m2 - m1)
    d = 1.0 + e2
    ew_ref[...] = jnp.concatenate([1.0 / d, e2 / d], axis=-1)

    p = jnp.exp(logits - m1)
    denom = jnp.sum(p, axis=-1, keepdims=True)
    probs = p / denom
    load_acc[...] += jnp.sum(probs, axis=0, keepdims=True)
    ent_acc[0] += -jnp.sum(probs * jnp.log(probs + 1e-8))

    @pl.when(i == _NT - 1)
    def _finish():
        load = load_acc[...] / _TOKENS
        mean = jnp.mean(load)
        var = jnp.sum((load - mean) ** 2) / (_EXPERTS - 1)
        stat_ref[0] = var
        stat_ref[1] = ent_acc[0] / _TOKENS


def kernel(hidden_states, router_weight):
    logits, idx, ew, stats = pl.pallas_call(
        _router_kernel,
        grid=(_NT,),
        in_specs=[
            pl.BlockSpec((_TM, _HIDDEN), lambda i: (i, 0)),
            pl.BlockSpec((_EXPERTS, _HIDDEN), lambda i: (0, 0)),
        ],
        out_specs=[
            pl.BlockSpec((_TM, _EXPERTS), lambda i: (i, 0)),
            pl.BlockSpec((_TM, 2), lambda i: (i, 0)),
            pl.BlockSpec((_TM, 2), lambda i: (i, 0)),
            pl.BlockSpec(memory_space=pltpu.SMEM),
        ],
        out_shape=[
            jax.ShapeDtypeStruct((_TOKENS, _EXPERTS), jnp.float32),
            jax.ShapeDtypeStruct((_TOKENS, 2), jnp.int32),
            jax.ShapeDtypeStruct((_TOKENS, 2), jnp.float32),
            jax.ShapeDtypeStruct((2,), jnp.float32),
        ],
        scratch_shapes=[
            pltpu.VMEM((1, _EXPERTS), jnp.float32),
            pltpu.SMEM((1,), jnp.float32),
        ],
    )(hidden_states, router_weight)
    return (logits, idx, ew, stats[0], stats[1])


# speedup vs baseline: 1.3296x; 1.0006x over previous
import jax
import jax.numpy as jnp
from jax.experimental import pallas as pl
from jax.experimental.pallas import tpu as pltpu

_HIDDEN = 4096
_EXPERTS = 64
_TOKENS = 8192
_TM = 512
_NT = _TOKENS // _TM


def _router_kernel(ha_ref, hb_ref, w_ref, logits_ref, idx_ref, ew_ref,
                   stat_ref, load_acc, ent_acc):
    i = pl.program_id(0)

    @pl.when(i == 0)
    def _init():
        load_acc[...] = jnp.zeros_like(load_acc)
        ent_acc[0] = 0.0

    ha = ha_ref[...]
    hb = hb_ref[...]
    w = w_ref[...]
    logits = jax.lax.dot_general(
        ha, w[:, : _HIDDEN // 2], (((1,), (1,)), ((), ())),
        preferred_element_type=jnp.float32)
    logits = logits + jax.lax.dot_general(
        hb, w[:, _HIDDEN // 2 :], (((1,), (1,)), ((), ())),
        preferred_element_type=jnp.float32)
    logits_ref[...] = logits

    lane = jax.lax.broadcasted_iota(jnp.int32, (_TM, _EXPERTS), 1)
    big = jnp.int32(_EXPERTS)
    m1 = jnp.max(logits, axis=-1, keepdims=True)
    i1 = jnp.min(jnp.where(logits == m1, lane, big), axis=-1, keepdims=True)
    masked = jnp.where(lane == i1, -jnp.inf, logits)
    m2 = jnp.max(masked, axis=-1, keepdims=True)
    i2 = jnp.min(jnp.where(masked == m2, lane, big), axis=-1, keepdims=True)
    idx_ref[...] = jnp.concatenate([i1, i2], axis=-1)

    e2 = jnp.exp(m2 - m1)
    d = 1.0 + e2
    ew_ref[...] = jnp.concatenate([1.0 / d, e2 / d], axis=-1)

    p = jnp.exp(logits - m1)
    denom = jnp.sum(p, axis=-1, keepdims=True)
    probs = p / denom
    load_acc[...] += jnp.sum(probs, axis=0, keepdims=True)
    ent_acc[0] += -jnp.sum(probs * jnp.log(probs + 1e-8))

    @pl.when(i == _NT - 1)
    def _finish():
        load = load_acc[...] / _TOKENS
        mean = jnp.mean(load)
        var = jnp.sum((load - mean) ** 2) / (_EXPERTS - 1)
        stat_ref[0] = var
        stat_ref[1] = ent_acc[0] / _TOKENS


def kernel(hidden_states, router_weight):
    logits, idx, ew, stats = pl.pallas_call(
        _router_kernel,
        grid=(_NT,),
        in_specs=[
            pl.BlockSpec((_TM, _HIDDEN // 2), lambda i: (i, 0)),
            pl.BlockSpec((_TM, _HIDDEN // 2), lambda i: (i, 1)),
            pl.BlockSpec((_EXPERTS, _HIDDEN), lambda i: (0, 0)),
        ],
        out_specs=[
            pl.BlockSpec((_TM, _EXPERTS), lambda i: (i, 0)),
            pl.BlockSpec((_TM, 2), lambda i: (i, 0)),
            pl.BlockSpec((_TM, 2), lambda i: (i, 0)),
            pl.BlockSpec(memory_space=pltpu.SMEM),
        ],
        out_shape=[
            jax.ShapeDtypeStruct((_TOKENS, _EXPERTS), jnp.float32),
            jax.ShapeDtypeStruct((_TOKENS, 2), jnp.int32),
            jax.ShapeDtypeStruct((_TOKENS, 2), jnp.float32),
            jax.ShapeDtypeStruct((2,), jnp.float32),
        ],
        scratch_shapes=[
            pltpu.VMEM((1, _EXPERTS), jnp.float32),
            pltpu.SMEM((1,), jnp.float32),
        ],
    )(hidden_states, hidden_states, router_weight)
    return (logits, idx, ew, stats[0], stats[1])


# speedup vs baseline: 1.3848x; 1.0415x over previous
import jax
import jax.numpy as jnp
from jax.experimental import pallas as pl
from jax.experimental.pallas import tpu as pltpu

_HIDDEN = 4096
_EXPERTS = 64
_TOKENS = 8192
_TM = 1024
_NT = _TOKENS // _TM


def _router_kernel(h_ref, w_ref, logits_ref, idx_ref, ew_ref, stat_ref,
                   load_acc, ent_acc):
    i = pl.program_id(0)

    @pl.when(i == 0)
    def _init():
        load_acc[...] = jnp.zeros_like(load_acc)
        ent_acc[0] = 0.0

    h = h_ref[...]
    w = w_ref[...]
    logits = jax.lax.dot_general(
        h, w, (((1,), (1,)), ((), ())),
        preferred_element_type=jnp.float32)
    logits_ref[...] = logits

    lane = jax.lax.broadcasted_iota(jnp.int32, (_TM, _EXPERTS), 1)
    big = jnp.int32(_EXPERTS)
    m1 = jnp.max(logits, axis=-1, keepdims=True)
    i1 = jnp.min(jnp.where(logits == m1, lane, big), axis=-1, keepdims=True)
    masked = jnp.where(lane == i1, -jnp.inf, logits)
    m2 = jnp.max(masked, axis=-1, keepdims=True)
    i2 = jnp.min(jnp.where(masked == m2, lane, big), axis=-1, keepdims=True)
    idx_ref[...] = jnp.concatenate([i1, i2], axis=-1)

    e2 = jnp.exp(m2 - m1)
    d = 1.0 + e2
    ew_ref[...] = jnp.concatenate([1.0 / d, e2 / d], axis=-1)

    p = jnp.exp(logits - m1)
    denom = jnp.sum(p, axis=-1, keepdims=True)
    probs = p / denom
    load_acc[...] += jnp.sum(probs, axis=0, keepdims=True)
    ent_acc[0] += -jnp.sum(probs * jnp.log(probs + 1e-8))

    @pl.when(i == _NT - 1)
    def _finish():
        load = load_acc[...] / _TOKENS
        mean = jnp.mean(load)
        var = jnp.sum((load - mean) ** 2) / (_EXPERTS - 1)
        stat_ref[0] = var
        stat_ref[1] = ent_acc[0] / _TOKENS


def kernel(hidden_states, router_weight):
    logits, idx, ew, stats = pl.pallas_call(
        _router_kernel,
        grid=(_NT,),
        in_specs=[
            pl.BlockSpec((_TM, _HIDDEN), lambda i: (i, 0)),
            pl.BlockSpec((_EXPERTS, _HIDDEN), lambda i: (0, 0)),
        ],
        out_specs=[
            pl.BlockSpec((_TM, _EXPERTS), lambda i: (i, 0)),
            pl.BlockSpec((_TM, 2), lambda i: (i, 0)),
            pl.BlockSpec((_TM, 2), lambda i: (i, 0)),
            pl.BlockSpec(memory_space=pltpu.SMEM),
        ],
        out_shape=[
            jax.ShapeDtypeStruct((_TOKENS, _EXPERTS), jnp.float32),
            jax.ShapeDtypeStruct((_TOKENS, 2), jnp.int32),
            jax.ShapeDtypeStruct((_TOKENS, 2), jnp.float32),
            jax.ShapeDtypeStruct((2,), jnp.float32),
        ],
        scratch_shapes=[
            pltpu.VMEM((1, _EXPERTS), jnp.float32),
            pltpu.SMEM((1,), jnp.float32),
        ],
    )(hidden_states, router_weight)
    return (logits, idx, ew, stats[0], stats[1])
